# baseline (speedup 1.0000x reference)
import functools

import jax
import jax.numpy as jnp
from jax import lax
from jax.experimental import pallas as pl
from jax.experimental.pallas import tpu as pltpu
from jax.experimental.pallas import tpu_sc as plsc


def _make_sc_loss(B, H, W, n_pad):
    HW = H * W
    info = plsc.get_sparse_core_info()
    NC, NS, L = info.num_cores, info.num_subcores, info.num_lanes
    SPT = n_pad // NS
    CB = B // NC
    NPH = 2
    PB = CB // NPH
    RPT = PB * H // NS
    NBLK = RPT // 8
    mesh = plsc.VectorSubcoreMesh(core_axis_name="c", subcore_axis_name="s")

    @functools.partial(
        pl.kernel,
        out_type=jax.ShapeDtypeStruct((NC * NS, L), jnp.float32),
        mesh=mesh,
        scratch_types=[
            pltpu.VMEM_SHARED((PB * HW,), jnp.float32),
            pltpu.VMEM((8, W), jnp.float32),
            pltpu.VMEM((8, W), jnp.float32),
            pltpu.VMEM((SPT,), jnp.int32),
            pltpu.VMEM((SPT,), jnp.int32),
            pltpu.VMEM((SPT,), jnp.int32),
            pltpu.VMEM((PB, 128), jnp.int32),
            pltpu.VMEM((PB, 128), jnp.float32),
            pltpu.VMEM((CB * SPT,), jnp.float32),
            pltpu.VMEM((SPT,), jnp.float32),
            pltpu.VMEM((L,), jnp.float32),
            pltpu.SemaphoreType.DMA,
            pltpu.SemaphoreType.DMA,
            pltpu.SemaphoreType.DMA,
        ],
    )
    def sc_loss(pred_hbm, tgt_hbm, rows_hbm, cols_hbm, mask_hbm, parts_hbm,
                sp, tspa, tspb, rows_v, cols_v, sidx_v, fidx_v, g_v, tgt_v,
                mask_v, part_v, semc, semr, semg):
        cid = lax.axis_index("c")
        sid = lax.axis_index("s")
        wid = sid * NC + cid

        pltpu.sync_copy(rows_hbm.at[pl.ds(sid * SPT, SPT)], rows_v)
        pltpu.sync_copy(cols_hbm.at[pl.ds(sid * SPT, SPT)], cols_v)
        pltpu.sync_copy(mask_hbm.at[pl.ds(sid * SPT, SPT)], mask_v)
        pltpu.sync_copy(
            tgt_hbm.at[pl.ds(sid * (B * SPT) + cid * (CB * SPT), CB * SPT)],
            tgt_v)

        for c in range(SPT // L):
            r = rows_v[pl.ds(c * L, L)]
            cc = cols_v[pl.ds(c * L, L)]
            sidx_v[pl.ds(c * L, L)] = r * W + cc
        for li in range(PB):
            for c in range(SPT // L):
                fidx_v[li, pl.ds(c * L, L)] = \
                    sidx_v[pl.ds(c * L, L)] + li * HW

        tsp = [tspa, tspb]

        def stage_phase(p):
            base_row = (cid * CB + p * PB) * H + sid * RPT

            def block_src(blk):
                g8 = pl.multiple_of(base_row + blk * 8, 8)
                return pred_hbm.at[pl.ds(g8, 8), :]

            def fire_rows(blk, buf):
                sp_row = sid * RPT + blk * 8
                for rr in range(8):
                    pltpu.make_async_copy(
                        buf.at[rr, :],
                        sp.at[pl.ds((sp_row + rr) * W, W)], semr).start()

            def drain_rows(buf):
                for rr in range(8):
                    pltpu.make_async_copy(
                        buf.at[rr, :], sp.at[pl.ds(0, W)], semr).wait()

            blk0 = pltpu.make_async_copy(block_src(0), tsp[0], semc)
            blk0.start()
            for blk in range(NBLK):
                cur = tsp[blk % 2]
                pltpu.make_async_copy(block_src(blk), cur, semc).wait()
                fire_rows(blk, cur)
                if blk + 1 < NBLK:
                    nxt = tsp[(blk + 1) % 2]
                    if blk >= 1:
                        drain_rows(nxt)
                    pltpu.make_async_copy(block_src(blk + 1), nxt, semc).start()
            drain_rows(tsp[(NBLK - 2) % 2])
            drain_rows(tsp[(NBLK - 1) % 2])

        acc = jnp.zeros((L,), jnp.float32)
        for p in range(NPH):
            stage_phase(p)
            plsc.subcore_barrier()
            gathers = [
                pltpu.make_async_copy(sp.at[fidx_v.at[li]], g_v.at[li], semg)
                for li in range(PB)
            ]
            for cp in gathers:
                cp.start()
            for cp in gathers:
                cp.wait()
            for li in range(PB):
                bl = p * PB + li
                for c in range(SPT // L):
                    g = g_v[li, pl.ds(c * L, L)]
                    t = tgt_v[pl.ds(bl * SPT + c * L, L)]
                    acc = acc + jnp.abs(g - t) * mask_v[pl.ds(c * L, L)]
            plsc.subcore_barrier()

        part_v[...] = acc
        pltpu.sync_copy(part_v, parts_hbm.at[wid])

    return sc_loss


def _reduce_body(scale, parts_ref, out_ref):
    out_ref[...] = (jnp.sum(parts_ref[...]) * scale)[None, None]


def kernel(pred_images, target_runoff_values, station_rows, station_cols):
    B, _, H, W = pred_images.shape
    N = station_rows.shape[0]
    NS = 16
    SPT = -(-N // NS)
    SPT = -(-SPT // 128) * 128
    n_pad = SPT * NS

    pred2 = pred_images.reshape(B * H, W)
    rows_p = jnp.pad(station_rows, (0, n_pad - N))
    cols_p = jnp.pad(station_cols, (0, n_pad - N))
    tgt_p = jnp.pad(target_runoff_values[:, :B], ((0, n_pad - N), (0, 0)))
    tgt_prep = tgt_p.reshape(NS, SPT, B).transpose(0, 2, 1).reshape(-1)
    mask = (jnp.arange(n_pad, dtype=jnp.int32) < N).astype(jnp.float32)

    parts = _make_sc_loss(B, H, W, n_pad)(
        pred2, tgt_prep, rows_p, cols_p, mask)

    out = pl.pallas_call(
        functools.partial(_reduce_body, 1.0 / (B * N)),
        out_shape=jax.ShapeDtypeStruct((1, 1), jnp.float32),
    )(parts)
    return out[0, 0]


# speedup vs baseline: 1.4635x; 1.4635x over previous
import functools

import jax
import jax.numpy as jnp
from jax import lax
from jax.experimental import pallas as pl
from jax.experimental.pallas import tpu as pltpu
from jax.experimental.pallas import tpu_sc as plsc


def _make_sc_loss(B, H, W, n_pad):
    HW = H * W
    info = plsc.get_sparse_core_info()
    NC, NS, L = info.num_cores, info.num_subcores, info.num_lanes
    SPT = n_pad // NS
    CB = B // NC
    NPH = 4
    PB = CB // NPH
    RPT = PB * H // NS
    NBLK = RPT // 8
    mesh = plsc.VectorSubcoreMesh(core_axis_name="c", subcore_axis_name="s")

    @functools.partial(
        pl.kernel,
        out_type=jax.ShapeDtypeStruct((NC * NS, L), jnp.float32),
        mesh=mesh,
        scratch_types=[
            pltpu.VMEM_SHARED((PB * HW,), jnp.float32),
            pltpu.VMEM((RPT // 2, W), jnp.float32),
            pltpu.VMEM((RPT // 2, W), jnp.float32),
            pltpu.VMEM((SPT,), jnp.int32),
            pltpu.VMEM((SPT,), jnp.int32),
            pltpu.VMEM((SPT,), jnp.int32),
            pltpu.VMEM((PB, 128), jnp.int32),
            pltpu.VMEM((PB, 128), jnp.float32),
            pltpu.VMEM((CB * SPT,), jnp.float32),
            pltpu.VMEM((SPT,), jnp.float32),
            pltpu.VMEM((L,), jnp.float32),
            pltpu.SemaphoreType.DMA,
            pltpu.SemaphoreType.DMA,
            pltpu.SemaphoreType.DMA,
        ],
    )
    def sc_loss(pred_hbm, tgt_hbm, rows_hbm, cols_hbm, mask_hbm, parts_hbm,
                sp, tspa, tspb, rows_v, cols_v, sidx_v, fidx_v, g_v, tgt_v,
                mask_v, part_v, semc, semr, semg):
        cid = lax.axis_index("c")
        sid = lax.axis_index("s")
        wid = sid * NC + cid

        pltpu.sync_copy(rows_hbm.at[pl.ds(sid * SPT, SPT)], rows_v)
        pltpu.sync_copy(cols_hbm.at[pl.ds(sid * SPT, SPT)], cols_v)
        pltpu.sync_copy(mask_hbm.at[pl.ds(sid * SPT, SPT)], mask_v)
        pltpu.sync_copy(
            tgt_hbm.at[pl.ds(sid * (B * SPT) + cid * (CB * SPT), CB * SPT)],
            tgt_v)

        for c in range(SPT // L):
            r = rows_v[pl.ds(c * L, L)]
            cc = cols_v[pl.ds(c * L, L)]
            sidx_v[pl.ds(c * L, L)] = r * W + cc
        for li in range(PB):
            for c in range(SPT // L):
                fidx_v[li, pl.ds(c * L, L)] = \
                    sidx_v[pl.ds(c * L, L)] + li * HW

        tsp = [tspa, tspb]
        HB = RPT // 2

        def block_copy(p, i):
            base_row = (cid * CB + p * PB) * H + sid * RPT
            g8 = pl.multiple_of(base_row + i * HB, 8)
            return pltpu.make_async_copy(
                pred_hbm.at[pl.ds(g8, HB), :], tsp[i], semc)

        def start_blocks(p):
            for i in range(2):
                block_copy(p, i).start()

        def forward_phase(p):
            for i in range(2):
                block_copy(p, i).wait()
                sp_base = sid * RPT + i * HB
                for rr in range(HB):
                    pltpu.make_async_copy(
                        tsp[i].at[rr, :],
                        sp.at[pl.ds((sp_base + rr) * W, W)], semr).start()
            for i in range(2):
                pltpu.make_async_copy(
                    pred_hbm.at[pl.ds(0, HB), :], tsp[i], semr).wait()

        acc = jnp.zeros((L,), jnp.float32)
        start_blocks(0)
        for p in range(NPH):
            forward_phase(p)
            if p + 1 < NPH:
                start_blocks(p + 1)
            plsc.subcore_barrier()
            gathers = [
                pltpu.make_async_copy(sp.at[fidx_v.at[li]], g_v.at[li], semg)
                for li in range(PB)
            ]
            for cp in gathers:
                cp.start()
            for cp in gathers:
                cp.wait()
            for li in range(PB):
                bl = p * PB + li
                for c in range(SPT // L):
                    g = g_v[li, pl.ds(c * L, L)]
                    t = tgt_v[pl.ds(bl * SPT + c * L, L)]
                    acc = acc + jnp.abs(g - t) * mask_v[pl.ds(c * L, L)]
            plsc.subcore_barrier()

        part_v[...] = acc
        pltpu.sync_copy(part_v, parts_hbm.at[wid])

    return sc_loss


def _reduce_body(scale, parts_ref, out_ref):
    out_ref[...] = (jnp.sum(parts_ref[...]) * scale)[None, None]


def kernel(pred_images, target_runoff_values, station_rows, station_cols):
    B, _, H, W = pred_images.shape
    N = station_rows.shape[0]
    NS = 16
    SPT = -(-N // NS)
    SPT = -(-SPT // 128) * 128
    n_pad = SPT * NS

    pred2 = pred_images.reshape(B * H, W)
    rows_p = jnp.pad(station_rows, (0, n_pad - N))
    cols_p = jnp.pad(station_cols, (0, n_pad - N))
    tgt_p = jnp.pad(target_runoff_values[:, :B], ((0, n_pad - N), (0, 0)))
    tgt_prep = tgt_p.reshape(NS, SPT, B).transpose(0, 2, 1).reshape(-1)
    mask = (jnp.arange(n_pad, dtype=jnp.int32) < N).astype(jnp.float32)

    parts = _make_sc_loss(B, H, W, n_pad)(
        pred2, tgt_prep, rows_p, cols_p, mask)

    out = pl.pallas_call(
        functools.partial(_reduce_body, 1.0 / (B * N)),
        out_shape=jax.ShapeDtypeStruct((1, 1), jnp.float32),
    )(parts)
    return out[0, 0]
